# Initial kernel scaffold; baseline (speedup 1.0000x reference)
import jax
import jax.numpy as jnp
from jax.experimental import pallas as pl


def kernel(x, Wg, bg, W1, b1, W2, b2):
    raise NotImplementedError("write your pallas kernel here")


# speedup vs baseline: 1.6765x; 1.6765x over previous
import functools

import jax
import jax.numpy as jnp
from jax import lax
from jax.experimental import pallas as pl
from jax.experimental.pallas import tpu as pltpu

E = 64
D = 768
DFF = 3072
T = 2048
B = 64
NT = T // B + E


def _gate_body(x_ref, wg_ref, bg_ref, eidx_ref, wgt_ref, aux_ref):
    x = x_ref[...]
    wg = wg_ref[...]
    logits = lax.dot_general(
        x, wg, (((1,), (1,)), ((), ())),
        preferred_element_type=jnp.float32,
    ) + bg_ref[...]
    m = jnp.max(logits, axis=1, keepdims=True)
    ex = jnp.exp(logits - m)
    probs = ex / jnp.sum(ex, axis=1, keepdims=True)

    pm = jnp.max(probs, axis=1, keepdims=True)
    iot = lax.broadcasted_iota(jnp.int32, (T, E), 1)
    idx = jnp.min(jnp.where(probs == pm, iot, E), axis=1)

    eidx_ref[...] = idx
    wgt_ref[...] = pm[:, 0]

    importance = jnp.sum(probs, axis=0)
    mu = jnp.sum(importance) / E
    imp_loss = jnp.sum((importance - mu) ** 2) / (E - 1) / (E * E)
    onehot = (iot == idx[:, None]).astype(jnp.float32)
    counts = jnp.sum(onehot, axis=0)
    wsum = jnp.sum(pm * onehot, axis=0)
    lb = E * jnp.sum((counts / T) * (wsum / T))
    aux_ref[...] = jnp.full((1, 1), imp_loss + lb, jnp.float32)


def _gate(x, Wg, bg):
    return pl.pallas_call(
        _gate_body,
        out_shape=(
            jax.ShapeDtypeStruct((T,), jnp.int32),
            jax.ShapeDtypeStruct((T,), jnp.float32),
            jax.ShapeDtypeStruct((1, 1), jnp.float32),
        ),
    )(x, Wg, bg.reshape(1, E))


def _ffn_body(te_ref, nreal_ref, xs_ref, ws_ref, w1_ref, b1_ref, w2_ref,
              b2_ref, out_ref):
    t = pl.program_id(0)

    @pl.when(t < nreal_ref[0])
    def _():
        xt = xs_ref[...]
        h = lax.dot_general(
            xt, w1_ref[0], (((1,), (1,)), ((), ())),
            preferred_element_type=jnp.float32,
            precision=lax.Precision.HIGHEST)
        h = jnp.maximum(h + b1_ref[0], 0.0)
        y = lax.dot_general(
            h, w2_ref[0], (((1,), (1,)), ((), ())),
            preferred_element_type=jnp.float32,
            precision=lax.Precision.HIGHEST)
        y = y + b2_ref[0]
        out_ref[...] = y * ws_ref[0, 0, :][:, None]


def _ffn(xs, ws3, W1, b1, W2, b2, tile_expert, nreal):
    grid_spec = pltpu.PrefetchScalarGridSpec(
        num_scalar_prefetch=2,
        grid=(NT,),
        in_specs=[
            pl.BlockSpec((B, D), lambda t, te, nr: (t, 0)),
            pl.BlockSpec((1, 1, B), lambda t, te, nr: (t, 0, 0)),
            pl.BlockSpec((1, DFF, D), lambda t, te, nr: (te[t], 0, 0)),
            pl.BlockSpec((1, 1, DFF), lambda t, te, nr: (te[t], 0, 0)),
            pl.BlockSpec((1, D, DFF), lambda t, te, nr: (te[t], 0, 0)),
            pl.BlockSpec((1, 1, D), lambda t, te, nr: (te[t], 0, 0)),
        ],
        out_specs=pl.BlockSpec((B, D), lambda t, te, nr: (t, 0)),
    )
    return pl.pallas_call(
        _ffn_body,
        grid_spec=grid_spec,
        out_shape=jax.ShapeDtypeStruct((NT * B, D), jnp.float32),
    )(tile_expert, nreal, xs, ws3, W1, b1.reshape(E, 1, DFF), W2,
      b2.reshape(E, 1, D))


def kernel(x, Wg, bg, W1, b1, W2, b2):
    eidx, wgt, aux = _gate(x, Wg, bg)

    counts = jnp.sum((eidx[:, None] == jnp.arange(E, dtype=jnp.int32)
                      ).astype(jnp.int32), axis=0)
    tiles_e = (counts + (B - 1)) // B
    tile_end = jnp.cumsum(tiles_e)
    nreal = tile_end[E - 1]
    pad_off = (tile_end - tiles_e) * B
    cum_counts = jnp.cumsum(counts) - counts

    order = jnp.argsort(eidx)
    es = eidx[order]
    rank = jnp.arange(T, dtype=jnp.int32) - cum_counts[es]
    pos_sorted = pad_off[es] + rank
    perm = jnp.zeros((NT * B,), jnp.int32).at[pos_sorted].set(order)
    pos_token = jnp.zeros((T,), jnp.int32).at[order].set(pos_sorted)

    te_raw = jnp.searchsorted(tile_end, jnp.arange(NT, dtype=jnp.int32),
                              side="right").astype(jnp.int32)
    te_last = jnp.max(es).astype(jnp.int32)
    tile_expert = jnp.where(jnp.arange(NT) < nreal, te_raw, te_last)

    xs = x[perm]
    ws = jnp.zeros((NT * B,), jnp.float32).at[pos_sorted].set(wgt[order])
    ws3 = ws.reshape(NT, 1, B)

    out_s = _ffn(xs, ws3, W1, b1, W2, b2, tile_expert,
                 nreal.reshape(1).astype(jnp.int32))
    out = out_s[pos_token]
    return out, aux.reshape(())


# speedup vs baseline: 3.8281x; 2.2834x over previous
import functools

import jax
import jax.numpy as jnp
from jax import lax
from jax.experimental import pallas as pl
from jax.experimental.pallas import tpu as pltpu

E = 64
D = 768
DFF = 3072
T = 2048
B = 64
NT = T // B + E


def _gate_body(x_ref, wg_ref, bg_ref, eidx_ref, wgt_ref, aux_ref):
    x = x_ref[...]
    wg = wg_ref[...]
    logits = lax.dot_general(
        x, wg, (((1,), (1,)), ((), ())),
        preferred_element_type=jnp.float32,
    ) + bg_ref[...]
    m = jnp.max(logits, axis=1, keepdims=True)
    ex = jnp.exp(logits - m)
    probs = ex / jnp.sum(ex, axis=1, keepdims=True)

    pm = jnp.max(probs, axis=1, keepdims=True)
    iot = lax.broadcasted_iota(jnp.int32, (T, E), 1)
    idx = jnp.min(jnp.where(probs == pm, iot, E), axis=1)

    eidx_ref[...] = idx
    wgt_ref[...] = pm[:, 0]

    importance = jnp.sum(probs, axis=0)
    mu = jnp.sum(importance) / E
    imp_loss = jnp.sum((importance - mu) ** 2) / (E - 1) / (E * E)
    onehot = (iot == idx[:, None]).astype(jnp.float32)
    counts = jnp.sum(onehot, axis=0)
    wsum = jnp.sum(pm * onehot, axis=0)
    lb = E * jnp.sum((counts / T) * (wsum / T))
    aux_ref[...] = jnp.full((1, 1), imp_loss + lb, jnp.float32)


def _gate(x, Wg, bg):
    return pl.pallas_call(
        _gate_body,
        out_shape=(
            jax.ShapeDtypeStruct((T,), jnp.int32),
            jax.ShapeDtypeStruct((T,), jnp.float32),
            jax.ShapeDtypeStruct((1, 1), jnp.float32),
        ),
    )(x, Wg, bg.reshape(1, E))


def _ffn_body(te_ref, nreal_ref, xs_ref, ws_ref, w1_ref, b1_ref, w2_ref,
              b2_ref, out_ref):
    t = pl.program_id(0)

    @pl.when(t < nreal_ref[0])
    def _():
        xt = xs_ref[...]
        h = lax.dot_general(
            xt, w1_ref[0], (((1,), (1,)), ((), ())),
            preferred_element_type=jnp.float32)
        h = jnp.maximum(h + b1_ref[0], 0.0)
        y = lax.dot_general(
            h, w2_ref[0], (((1,), (1,)), ((), ())),
            preferred_element_type=jnp.float32)
        y = y + b2_ref[0]
        out_ref[...] = y * ws_ref[0, 0, :][:, None]


def _ffn(xs, ws3, W1, b1, W2, b2, tile_expert, nreal):
    grid_spec = pltpu.PrefetchScalarGridSpec(
        num_scalar_prefetch=2,
        grid=(NT,),
        in_specs=[
            pl.BlockSpec((B, D), lambda t, te, nr: (t, 0)),
            pl.BlockSpec((1, 1, B), lambda t, te, nr: (t, 0, 0)),
            pl.BlockSpec((1, DFF, D), lambda t, te, nr: (te[t], 0, 0)),
            pl.BlockSpec((1, 1, DFF), lambda t, te, nr: (te[t], 0, 0)),
            pl.BlockSpec((1, D, DFF), lambda t, te, nr: (te[t], 0, 0)),
            pl.BlockSpec((1, 1, D), lambda t, te, nr: (te[t], 0, 0)),
        ],
        out_specs=pl.BlockSpec((B, D), lambda t, te, nr: (t, 0)),
    )
    return pl.pallas_call(
        _ffn_body,
        grid_spec=grid_spec,
        out_shape=jax.ShapeDtypeStruct((NT * B, D), jnp.float32),
    )(tile_expert, nreal, xs, ws3, W1, b1.reshape(E, 1, DFF), W2,
      b2.reshape(E, 1, D))


def kernel(x, Wg, bg, W1, b1, W2, b2):
    eidx, wgt, aux = _gate(x, Wg, bg)

    counts = jnp.sum((eidx[:, None] == jnp.arange(E, dtype=jnp.int32)
                      ).astype(jnp.int32), axis=0)
    tiles_e = (counts + (B - 1)) // B
    tile_end = jnp.cumsum(tiles_e)
    nreal = tile_end[E - 1]
    pad_off = (tile_end - tiles_e) * B
    cum_counts = jnp.cumsum(counts) - counts

    order = jnp.argsort(eidx)
    es = eidx[order]
    rank = jnp.arange(T, dtype=jnp.int32) - cum_counts[es]
    pos_sorted = pad_off[es] + rank
    perm = jnp.zeros((NT * B,), jnp.int32).at[pos_sorted].set(order)
    pos_token = jnp.zeros((T,), jnp.int32).at[order].set(pos_sorted)

    te_raw = jnp.searchsorted(tile_end, jnp.arange(NT, dtype=jnp.int32),
                              side="right").astype(jnp.int32)
    te_last = jnp.max(es).astype(jnp.int32)
    tile_expert = jnp.where(jnp.arange(NT) < nreal, te_raw, te_last)

    xs = x[perm]
    ws = jnp.zeros((NT * B,), jnp.float32).at[pos_sorted].set(wgt[order])
    ws3 = ws.reshape(NT, 1, B)

    out_s = _ffn(xs, ws3, W1, b1, W2, b2, tile_expert,
                 nreal.reshape(1).astype(jnp.int32))
    out = out_s[pos_token]
    return out, aux.reshape(())
